# baseline (speedup 1.0000x reference)
import functools

import jax
import jax.numpy as jnp
from jax import lax
from jax.experimental import pallas as pl
from jax.experimental.pallas import tpu as pltpu
from jax.experimental.pallas import tpu_sc as plsc

_NC = 2
_NS = 16
_NW = _NC * _NS
_N_ROWS = 4 * 8192
_ROWS_W = _N_ROWS // _NW
_CHUNKS = _ROWS_W // 16

_LN2 = 0.6931471805599453


def _vlog(x):
    bits = plsc.bitcast(x, jnp.int32)
    e = lax.shift_right_logical(bits, jnp.int32(23)) - jnp.int32(126)
    mbits = (bits & jnp.int32(0x007FFFFF)) | jnp.int32(0x3F000000)
    m = plsc.bitcast(mbits, jnp.float32)
    small = m < jnp.float32(0.70710678)
    m2 = jnp.where(small, m * jnp.float32(2.0), m)
    ef = e.astype(jnp.float32) - jnp.where(small, jnp.float32(1.0),
                                           jnp.float32(0.0))
    t = (m2 - jnp.float32(1.0)) / (m2 + jnp.float32(1.0))
    w = t * t
    r = t * (jnp.float32(2.0) + w * (jnp.float32(2.0 / 3.0)
             + w * (jnp.float32(2.0 / 5.0) + w * jnp.float32(2.0 / 7.0))))
    return ef * jnp.float32(_LN2) + r


def _mi_body(logits_hbm, masks_hbm, out_hbm, lv, mv, ov):
    wid = lax.axis_index("s") * _NC + lax.axis_index("c")
    pltpu.sync_copy(logits_hbm.at[pl.ds(wid * _ROWS_W * 3, _ROWS_W * 3)], lv)
    pltpu.sync_copy(masks_hbm.at[pl.ds(wid * _ROWS_W, _ROWS_W)], mv)

    def body(i, carry):
        acc_h, acc_p0, acc_p12, acc_c = carry
        base = i * jnp.int32(48)
        idx = lax.iota(jnp.int32, 16) * jnp.int32(3) + base
        a0 = plsc.load_gather(lv, [idx])
        a1 = plsc.load_gather(lv, [idx + jnp.int32(1)])
        a2 = plsc.load_gather(lv, [idx + jnp.int32(2)])
        mraw = mv[pl.ds(i * 16, 16)]
        mf = jnp.where(mraw != jnp.int32(0), jnp.float32(1.0),
                       jnp.float32(0.0))
        mx = jnp.maximum(a0, jnp.maximum(a1, a2))
        e0 = jnp.exp(a0 - mx)
        e1 = jnp.exp(a1 - mx)
        e2 = jnp.exp(a2 - mx)
        sinv = jnp.float32(1.0) / (e0 + e1 + e2)
        p0 = e0 * sinv
        p12 = (e1 + e2) * sinv
        h = -(p0 * _vlog(p0) + p12 * _vlog(p12))
        return (acc_h + h * mf, acc_p0 + p0 * mf,
                acc_p12 + p12 * mf, acc_c + mf)

    z = jnp.zeros((16,), jnp.float32)
    acc_h, acc_p0, acc_p12, acc_c = lax.fori_loop(
        0, _CHUNKS, body, (z, z, z, z))

    ov[0, :] = acc_h
    ov[1, :] = acc_p0
    ov[2, :] = acc_p12
    ov[3, :] = acc_c
    pltpu.sync_copy(ov, out_hbm.at[wid])


_mi_kernel = functools.partial(
    pl.kernel,
    mesh=plsc.VectorSubcoreMesh(core_axis_name="c", subcore_axis_name="s"),
    out_type=jax.ShapeDtypeStruct((_NW, 4, 16), jnp.float32),
    scratch_types=[
        pltpu.VMEM((_ROWS_W * 3,), jnp.float32),
        pltpu.VMEM((_ROWS_W,), jnp.int32),
        pltpu.VMEM((4, 16), jnp.float32),
    ],
    compiler_params=pltpu.CompilerParams(needs_layout_passes=False),
)(_mi_body)


def kernel(logits, masks):
    lf = logits.reshape(-1)
    mf = masks.reshape(-1).astype(jnp.int32)
    part = _mi_kernel(lf, mf)
    sums = jnp.sum(part, axis=(0, 2))
    count = sums[3]
    condi = sums[0] / count
    y0 = sums[1] / count
    y1 = sums[2] / count
    y_entropy = -(y0 * jnp.log(y0) + y1 * jnp.log(y1))
    first = jnp.where(y_entropy < jnp.float32(0.5),
                      -y_entropy + condi, condi)
    return (first, y_entropy)


# speedup vs baseline: 1.0219x; 1.0219x over previous
import functools

import jax
import jax.numpy as jnp
from jax import lax
from jax.experimental import pallas as pl
from jax.experimental.pallas import tpu as pltpu
from jax.experimental.pallas import tpu_sc as plsc

_NC = 1
_NS = 16
_NW = _NC * _NS
_N_ROWS = 4 * 8192
_ROWS_W = _N_ROWS // _NW
_CHUNKS = _ROWS_W // 16

_LN2 = 0.6931471805599453


def _vlog(x):
    bits = plsc.bitcast(x, jnp.int32)
    e = lax.shift_right_logical(bits, jnp.int32(23)) - jnp.int32(126)
    mbits = (bits & jnp.int32(0x007FFFFF)) | jnp.int32(0x3F000000)
    m = plsc.bitcast(mbits, jnp.float32)
    small = m < jnp.float32(0.70710678)
    m2 = jnp.where(small, m * jnp.float32(2.0), m)
    ef = e.astype(jnp.float32) - jnp.where(small, jnp.float32(1.0),
                                           jnp.float32(0.0))
    t = (m2 - jnp.float32(1.0)) / (m2 + jnp.float32(1.0))
    w = t * t
    r = t * (jnp.float32(2.0) + w * (jnp.float32(2.0 / 3.0)
             + w * (jnp.float32(2.0 / 5.0) + w * jnp.float32(2.0 / 7.0))))
    return ef * jnp.float32(_LN2) + r


def _mi_body(logits_hbm, masks_hbm, out_hbm, lv, mv, ov):
    wid = lax.axis_index("s") * _NC + lax.axis_index("c")
    pltpu.sync_copy(logits_hbm.at[pl.ds(wid * _ROWS_W * 3, _ROWS_W * 3)], lv)
    pltpu.sync_copy(masks_hbm.at[pl.ds(wid * _ROWS_W, _ROWS_W)], mv)

    def body(i, carry):
        acc_h, acc_p0, acc_p12, acc_c = carry
        base = i * jnp.int32(48)
        idx = lax.iota(jnp.int32, 16) * jnp.int32(3) + base
        a0 = plsc.load_gather(lv, [idx])
        a1 = plsc.load_gather(lv, [idx + jnp.int32(1)])
        a2 = plsc.load_gather(lv, [idx + jnp.int32(2)])
        mraw = mv[pl.ds(i * 16, 16)]
        mf = jnp.where(mraw != jnp.int32(0), jnp.float32(1.0),
                       jnp.float32(0.0))
        mx = jnp.maximum(a0, jnp.maximum(a1, a2))
        e0 = jnp.exp(a0 - mx)
        e1 = jnp.exp(a1 - mx)
        e2 = jnp.exp(a2 - mx)
        sinv = jnp.float32(1.0) / (e0 + e1 + e2)
        p0 = e0 * sinv
        p12 = (e1 + e2) * sinv
        h = -(p0 * _vlog(p0) + p12 * _vlog(p12))
        return (acc_h + h * mf, acc_p0 + p0 * mf,
                acc_p12 + p12 * mf, acc_c + mf)

    z = jnp.zeros((16,), jnp.float32)
    acc_h, acc_p0, acc_p12, acc_c = lax.fori_loop(
        0, _CHUNKS, body, (z, z, z, z))

    ov[0, :] = acc_h
    ov[1, :] = acc_p0
    ov[2, :] = acc_p12
    ov[3, :] = acc_c
    pltpu.sync_copy(ov, out_hbm.at[wid])


_mi_kernel = functools.partial(
    pl.kernel,
    mesh=plsc.VectorSubcoreMesh(core_axis_name="c", subcore_axis_name="s", num_cores=1),
    out_type=jax.ShapeDtypeStruct((_NW, 4, 16), jnp.float32),
    scratch_types=[
        pltpu.VMEM((_ROWS_W * 3,), jnp.float32),
        pltpu.VMEM((_ROWS_W,), jnp.int32),
        pltpu.VMEM((4, 16), jnp.float32),
    ],
    compiler_params=pltpu.CompilerParams(needs_layout_passes=False,
                                         skip_device_barrier=True),
)(_mi_body)


def kernel(logits, masks):
    lf = logits.reshape(-1)
    mf = masks.reshape(-1).astype(jnp.int32)
    part = _mi_kernel(lf, mf)
    sums = jnp.sum(part, axis=(0, 2))
    count = sums[3]
    condi = sums[0] / count
    y0 = sums[1] / count
    y1 = sums[2] / count
    y_entropy = -(y0 * jnp.log(y0) + y1 * jnp.log(y1))
    first = jnp.where(y_entropy < jnp.float32(0.5),
                      -y_entropy + condi, condi)
    return (first, y_entropy)


# speedup vs baseline: 9.0809x; 8.8865x over previous
import jax
import jax.numpy as jnp
from jax.experimental import pallas as pl

_R = 256
_C = 128


def _mi_tc_body(a0, a1, a2, mk, out_first, out_ye):
    x0 = a0[...]
    x1 = a1[...]
    x2 = a2[...]
    mf = jnp.where(mk[...] != 0, jnp.float32(1.0), jnp.float32(0.0))
    mx = jnp.maximum(x0, jnp.maximum(x1, x2))
    e0 = jnp.exp(x0 - mx)
    e1 = jnp.exp(x1 - mx)
    e2 = jnp.exp(x2 - mx)
    sinv = jnp.float32(1.0) / (e0 + e1 + e2)
    p0 = e0 * sinv
    p12 = (e1 + e2) * sinv
    h = -(p0 * jnp.log(p0) + p12 * jnp.log(p12))
    count = jnp.sum(mf)
    cinv = jnp.float32(1.0) / count
    condi = jnp.sum(h * mf) * cinv
    y0 = jnp.sum(p0 * mf) * cinv
    y1 = jnp.sum(p12 * mf) * cinv
    ye = -(y0 * jnp.log(y0) + y1 * jnp.log(y1))
    first = jnp.where(ye < jnp.float32(0.5), condi - ye, condi)
    out_first[...] = jnp.broadcast_to(first, (1, 1))
    out_ye[...] = jnp.broadcast_to(ye, (1, 1))


def kernel(logits, masks):
    lt = logits.reshape(-1, 3)
    a0 = lt[:, 0].reshape(_R, _C)
    a1 = lt[:, 1].reshape(_R, _C)
    a2 = lt[:, 2].reshape(_R, _C)
    mk = masks.reshape(_R, _C).astype(jnp.int32)
    first, ye = pl.pallas_call(
        _mi_tc_body,
        out_shape=(jax.ShapeDtypeStruct((1, 1), jnp.float32),
                   jax.ShapeDtypeStruct((1, 1), jnp.float32)),
    )(a0, a1, a2, mk)
    return (first[0, 0], ye[0, 0])
